# Initial kernel scaffold; baseline (speedup 1.0000x reference)
import jax
import jax.numpy as jnp
from jax.experimental import pallas as pl


def kernel(x, h, c, edge_index, time, W_iou, U_iou, b_iou, U_f, W_f, b_f):
    raise NotImplementedError("write your pallas kernel here")


# speedup vs baseline: 1.0222x; 1.0222x over previous
import functools

import jax
import jax.numpy as jnp
from jax import lax
from jax.experimental import pallas as pl
from jax.experimental.pallas import tpu as pltpu
from jax.experimental.pallas import tpu_sc as plsc

N = 10000
E = 320000
H = 128
NTILES = 16
CHUNK = 80
EDGES_PER_TILE = E // NTILES
NCHUNKS = EDGES_PER_TILE // CHUNK
STRIPE = 624
TAIL = N - NTILES * STRIPE
TAIL_BASE = NTILES * STRIPE
BN = 1000


def _tc1_body(x_ref, h_ref, wft_ref, uft_ref, wiout_ref, bf_ref, biou_ref,
              wxf_ref, hu_ref, xwiou_ref, s_ref):
    x = x_ref[...]
    h = h_ref[...]
    wxf_ref[...] = (
        jnp.dot(x, wft_ref[...], preferred_element_type=jnp.float32)
        + bf_ref[...])
    hu_ref[...] = jnp.dot(h, uft_ref[...], preferred_element_type=jnp.float32)
    xwiou_ref[...] = (
        jnp.dot(x, wiout_ref[...], preferred_element_type=jnp.float32)
        + biou_ref[...])
    s_ref[...] = jnp.sum(uft_ref[...], axis=0, keepdims=True)


def _tc1(x, h, wft, uft, wiout, bf, biou):
    grid = (N // BN,)
    return pl.pallas_call(
        _tc1_body,
        grid=grid,
        in_specs=[
            pl.BlockSpec((BN, H), lambda i: (i, 0)),
            pl.BlockSpec((BN, H), lambda i: (i, 0)),
            pl.BlockSpec((H, H), lambda i: (0, 0)),
            pl.BlockSpec((H, H), lambda i: (0, 0)),
            pl.BlockSpec((H, 3 * H), lambda i: (0, 0)),
            pl.BlockSpec((1, H), lambda i: (0, 0)),
            pl.BlockSpec((1, 3 * H), lambda i: (0, 0)),
        ],
        out_specs=[
            pl.BlockSpec((BN, H), lambda i: (i, 0)),
            pl.BlockSpec((BN, H), lambda i: (i, 0)),
            pl.BlockSpec((BN, 3 * H), lambda i: (i, 0)),
            pl.BlockSpec((1, H), lambda i: (0, 0)),
        ],
        out_shape=[
            jax.ShapeDtypeStruct((N, H), jnp.float32),
            jax.ShapeDtypeStruct((N, H), jnp.float32),
            jax.ShapeDtypeStruct((N, 3 * H), jnp.float32),
            jax.ShapeDtypeStruct((1, H), jnp.float32),
        ],
    )(x, h, wft, uft, wiout, bf, biou)


_sc_mesh = plsc.VectorSubcoreMesh(core_axis_name="c", subcore_axis_name="s")


@functools.partial(
    pl.kernel,
    out_type=jax.ShapeDtypeStruct((2 * N, H), jnp.float32),
    mesh=_sc_mesh,
    scratch_types=[
        pltpu.VMEM((CHUNK,), jnp.int32),
        pltpu.VMEM((CHUNK,), jnp.int32),
        pltpu.VMEM((CHUNK,), jnp.int32),
        pltpu.VMEM((CHUNK, 16), jnp.float32),
        pltpu.VMEM((CHUNK, 64), jnp.float32),
        pltpu.VMEM((CHUNK, 64), jnp.float32),
        pltpu.VMEM((CHUNK, 64), jnp.float32),
        pltpu.VMEM((CHUNK, 64), jnp.float32),
        pltpu.VMEM((CHUNK, H), jnp.float32),
        pltpu.VMEM((64,), jnp.float32),
        pltpu.VMEM_SHARED((N, H), jnp.float32),
        pltpu.SemaphoreType.DMA,
        pltpu.SemaphoreType.DMA,
        pltpu.SemaphoreType.DMA,
        pltpu.SemaphoreType.DMA,
    ],
    compiler_params=pltpu.CompilerParams(use_tc_tiling_on_sc=False),
)
def _sc_edge_kernel(hu2, c2, h2, w2, src, dst, t16, svec, zeros, out,
                    src_v, dst_v, dsto_v, time_v, g_hu, g_c, g_h, g_w,
                    out_v, s_v, acc, sem0, sem1, sem2, sem3):
    cid = lax.axis_index("c")
    sid = lax.axis_index("s")

    row0 = sid * STRIPE
    pltpu.sync_copy(zeros.at[pl.ds(row0, STRIPE)],
                    acc.at[pl.ds(row0, STRIPE)])

    @pl.when(sid == NTILES - 1)
    def _zero_tail():
        pltpu.sync_copy(zeros.at[pl.ds(TAIL_BASE, TAIL)],
                        acc.at[pl.ds(TAIL_BASE, TAIL)])
    pltpu.sync_copy(svec.at[pl.ds(cid * 64, 64)], s_v)
    plsc.subcore_barrier()

    ebase = sid * EDGES_PER_TILE

    def chunk_body(ci, carry):
        base = ebase + ci * CHUNK
        pltpu.sync_copy(src.at[pl.ds(base, CHUNK)], src_v)
        pltpu.sync_copy(dst.at[pl.ds(base, CHUNK)], dst_v)
        pltpu.sync_copy(t16.at[pl.ds(base, CHUNK)], time_v)
        for j in range(CHUNK // 16):
            sl = pl.ds(j * 16, 16)
            src_v[sl] = src_v[sl] * 2 + cid
            dsto_v[sl] = dst_v[sl] * 2 + cid
        cp0 = pltpu.async_copy(hu2.at[src_v], g_hu, sem0)
        cp1 = pltpu.async_copy(c2.at[src_v], g_c, sem1)
        cp2 = pltpu.async_copy(h2.at[src_v], g_h, sem2)
        cp3 = pltpu.async_copy(w2.at[dsto_v], g_w, sem3)
        cp0.wait()
        cp1.wait()
        cp2.wait()
        cp3.wait()

        def edge_body(e, c2_):
            tv = time_v[e, pl.ds(0, 16)]
            for j in range(4):
                sl = pl.ds(j * 16, 16)
                z = g_hu[e, sl] + g_w[e, sl] + tv * s_v[sl]
                f = 1.0 / (1.0 + jnp.exp(-z))
                out_v[e, sl] = f * g_c[e, sl]
                out_v[e, pl.ds(64 + j * 16, 16)] = g_h[e, sl] + tv
            return c2_

        lax.fori_loop(0, CHUNK, edge_body, 0)
        pltpu.sync_copy(out_v, acc.at[dst_v], add=True)
        return carry

    lax.fori_loop(0, NCHUNKS, chunk_body, 0)
    plsc.subcore_barrier()
    pltpu.sync_copy(acc.at[pl.ds(row0, STRIPE)],
                    out.at[pl.ds(cid * N + row0, STRIPE)])

    @pl.when(sid == NTILES - 1)
    def _out_tail():
        pltpu.sync_copy(acc.at[pl.ds(TAIL_BASE, TAIL)],
                        out.at[pl.ds(cid * N + TAIL_BASE, TAIL)])


def _tc2_body(ht_ref, cred_ref, xwiou_ref, uiout_ref, hnew_ref, cnew_ref):
    iou = (jnp.dot(ht_ref[...], uiout_ref[...],
                   preferred_element_type=jnp.float32)
           + xwiou_ref[...])
    i = jax.nn.sigmoid(iou[:, :H])
    o = jax.nn.sigmoid(iou[:, H:2 * H])
    u = jnp.tanh(iou[:, 2 * H:])
    cn = i * u + cred_ref[...]
    cnew_ref[...] = cn
    hnew_ref[...] = o * jnp.tanh(cn)


def _tc2(h_tild, c_red, xwiou, uiout):
    grid = (N // BN,)
    return pl.pallas_call(
        _tc2_body,
        grid=grid,
        in_specs=[
            pl.BlockSpec((BN, H), lambda i: (i, 0)),
            pl.BlockSpec((BN, H), lambda i: (i, 0)),
            pl.BlockSpec((BN, 3 * H), lambda i: (i, 0)),
            pl.BlockSpec((H, 3 * H), lambda i: (0, 0)),
        ],
        out_specs=[
            pl.BlockSpec((BN, H), lambda i: (i, 0)),
            pl.BlockSpec((BN, H), lambda i: (i, 0)),
        ],
        out_shape=[
            jax.ShapeDtypeStruct((N, H), jnp.float32),
            jax.ShapeDtypeStruct((N, H), jnp.float32),
        ],
    )(h_tild, c_red, xwiou, uiout)


def kernel(x, h, c, edge_index, time, W_iou, U_iou, b_iou, U_f, W_f, b_f):
    x = x.astype(jnp.float32)
    h = h.astype(jnp.float32)
    c = c.astype(jnp.float32)
    src = edge_index[0].astype(jnp.int32)
    dst = edge_index[1].astype(jnp.int32)
    t16 = jnp.broadcast_to(time.astype(jnp.float32), (E, 16))

    wxf, hu, xwiou, s = _tc1(x, h, W_f.T, U_f.T, W_iou.T, b_f, b_iou)

    hu2 = hu.reshape(2 * N, 64)
    c2 = c.reshape(2 * N, 64)
    h2 = h.reshape(2 * N, 64)
    w2 = wxf.reshape(2 * N, 64)
    zeros = jnp.zeros((N, H), jnp.float32)

    accout = _sc_edge_kernel(hu2, c2, h2, w2, src, dst, t16,
                             s.reshape(H), zeros)
    acc3 = accout.reshape(2, N, H)
    c_red = jnp.concatenate([acc3[0, :, :64], acc3[1, :, :64]], axis=1)
    h_tild = jnp.concatenate([acc3[0, :, 64:], acc3[1, :, 64:]], axis=1)

    h_new, c_new = _tc2(h_tild, c_red, xwiou, U_iou.T)
    return (h_new, c_new)


# speedup vs baseline: 1.4024x; 1.3719x over previous
import functools

import jax
import jax.numpy as jnp
from jax import lax
from jax.experimental import pallas as pl
from jax.experimental.pallas import tpu as pltpu
from jax.experimental.pallas import tpu_sc as plsc

N = 10000
E = 320000
H = 128
NTILES = 16
CHUNK = 80
EDGES_PER_TILE = E // NTILES
NCHUNKS = EDGES_PER_TILE // CHUNK
STRIPE = 624
TAIL = N - NTILES * STRIPE
TAIL_BASE = NTILES * STRIPE
BN = 1000
NLIN = 4
NG = 2


def _tc1_body(x_ref, h_ref, c_ref, wft_ref, uft_ref, wiout_ref, bf_ref,
              biou_ref, wxf_ref, atab_ref, xwiou_ref, s_ref):
    x = x_ref[...]
    h = h_ref[...]
    cc = c_ref[...]
    wxf_ref[...] = (
        jnp.dot(x, wft_ref[...], preferred_element_type=jnp.float32)
        + bf_ref[...])
    hu = jnp.dot(h, uft_ref[...], preferred_element_type=jnp.float32)
    atab_ref[...] = jnp.concatenate(
        [hu[:, :64], cc[:, :64], hu[:, 64:], cc[:, 64:]], axis=1)
    xwiou_ref[...] = (
        jnp.dot(x, wiout_ref[...], preferred_element_type=jnp.float32)
        + biou_ref[...])
    s_ref[...] = jnp.sum(uft_ref[...], axis=0, keepdims=True)


def _tc1(x, h, c, wft, uft, wiout, bf, biou):
    grid = (N // BN,)
    return pl.pallas_call(
        _tc1_body,
        grid=grid,
        in_specs=[
            pl.BlockSpec((BN, H), lambda i: (i, 0)),
            pl.BlockSpec((BN, H), lambda i: (i, 0)),
            pl.BlockSpec((BN, H), lambda i: (i, 0)),
            pl.BlockSpec((H, H), lambda i: (0, 0)),
            pl.BlockSpec((H, H), lambda i: (0, 0)),
            pl.BlockSpec((H, 3 * H), lambda i: (0, 0)),
            pl.BlockSpec((1, H), lambda i: (0, 0)),
            pl.BlockSpec((1, 3 * H), lambda i: (0, 0)),
        ],
        out_specs=[
            pl.BlockSpec((BN, H), lambda i: (i, 0)),
            pl.BlockSpec((BN, 2 * H), lambda i: (i, 0)),
            pl.BlockSpec((BN, 3 * H), lambda i: (i, 0)),
            pl.BlockSpec((1, H), lambda i: (0, 0)),
        ],
        out_shape=[
            jax.ShapeDtypeStruct((N, H), jnp.float32),
            jax.ShapeDtypeStruct((N, 2 * H), jnp.float32),
            jax.ShapeDtypeStruct((N, 3 * H), jnp.float32),
            jax.ShapeDtypeStruct((1, H), jnp.float32),
        ],
    )(x, h, c, wft, uft, wiout, bf, biou)


_sc_mesh = plsc.VectorSubcoreMesh(core_axis_name="c", subcore_axis_name="s")

_sc_scratch = (
    [pltpu.VMEM((CHUNK,), jnp.int32) for _ in range(NLIN)]
    + [pltpu.VMEM((CHUNK,), jnp.int32) for _ in range(NLIN)]
    + [pltpu.VMEM((CHUNK,), jnp.int32) for _ in range(NG)]
    + [pltpu.VMEM((CHUNK, 16), jnp.float32) for _ in range(NLIN)]
    + [pltpu.VMEM((CHUNK, 2 * 64), jnp.float32) for _ in range(NG)]
    + [pltpu.VMEM((CHUNK, 64), jnp.float32) for _ in range(NG)]
    + [
        pltpu.VMEM((CHUNK, 64), jnp.float32),
        pltpu.VMEM((64,), jnp.float32),
        pltpu.VMEM_SHARED((N, 64), jnp.float32),
    ]
    + [pltpu.SemaphoreType.DMA for _ in range(NLIN + NG)]
)


def _sc_edge_body(atab, w2, h2, src, dst, t16, svec, zeros,
                  out_c, out_h, *scr):
    pos = 0
    lin_src = scr[pos:pos + NLIN]; pos += NLIN
    lin_dst = scr[pos:pos + NLIN]; pos += NLIN
    lin_dsto = scr[pos:pos + NG]; pos += NG
    lin_t = scr[pos:pos + NLIN]; pos += NLIN
    g_a = scr[pos:pos + NG]; pos += NG
    g_w = scr[pos:pos + NG]; pos += NG
    out_v, s_v, acc = scr[pos:pos + 3]; pos += 3
    sem_lin = scr[pos:pos + NLIN]; pos += NLIN
    sem_g = scr[pos:pos + NG]; pos += NG

    cid = lax.axis_index("c")
    sid = lax.axis_index("s")
    row0 = sid * STRIPE
    ebase = sid * EDGES_PER_TILE

    def zero_stripe():
        pltpu.sync_copy(zeros.at[pl.ds(row0, STRIPE)],
                        acc.at[pl.ds(row0, STRIPE)])

        @pl.when(sid == NTILES - 1)
        def _zero_tail():
            pltpu.sync_copy(zeros.at[pl.ds(TAIL_BASE, TAIL)],
                            acc.at[pl.ds(TAIL_BASE, TAIL)])

    def copy_out(out):
        pltpu.sync_copy(acc.at[pl.ds(row0, STRIPE)],
                        out.at[pl.ds(cid * N + row0, STRIPE)])

        @pl.when(sid == NTILES - 1)
        def _out_tail():
            pltpu.sync_copy(acc.at[pl.ds(TAIL_BASE, TAIL)],
                            out.at[pl.ds(cid * N + TAIL_BASE, TAIL)])

    def fire_lin(ci, l):
        base = ebase + ci * CHUNK
        pltpu.async_copy(src.at[pl.ds(base, CHUNK)], lin_src[l], sem_lin[l])
        pltpu.async_copy(dst.at[pl.ds(base, CHUNK)], lin_dst[l], sem_lin[l])
        pltpu.async_copy(t16.at[pl.ds(base, CHUNK)], lin_t[l], sem_lin[l])

    def wait_lin(l):
        pltpu.make_async_copy(src.at[pl.ds(0, CHUNK)], lin_src[l],
                              sem_lin[l]).wait()
        pltpu.make_async_copy(dst.at[pl.ds(0, CHUNK)], lin_dst[l],
                              sem_lin[l]).wait()
        pltpu.make_async_copy(t16.at[pl.ds(0, CHUNK)], lin_t[l],
                              sem_lin[l]).wait()

    def offset(ref_v):
        for j in range(CHUNK // 16):
            sl = pl.ds(j * 16, 16)
            ref_v[sl] = ref_v[sl] * 2 + cid

    def fire_gather_a(b, l):
        offset(lin_src[l])
        for j in range(CHUNK // 16):
            sl = pl.ds(j * 16, 16)
            lin_dsto[b][sl] = lin_dst[l][sl] * 2 + cid
        return [pltpu.async_copy(atab.at[lin_src[l]], g_a[b], sem_g[b]),
                pltpu.async_copy(w2.at[lin_dsto[b]], g_w[b], sem_g[b])]

    def wait_gather_a(b):
        pltpu.make_async_copy(atab.at[pl.ds(0, CHUNK)], g_a[b],
                              sem_g[b]).wait()
        pltpu.make_async_copy(w2.at[pl.ds(0, CHUNK)], g_w[b],
                              sem_g[b]).wait()

    def compute_a(b, l):
        def _edge(e, carry):
            tv = lin_t[l][e]
            for j in range(4):
                sl = pl.ds(j * 16, 16)
                z = g_a[b][e, sl] + g_w[b][e, sl] + tv * s_v[sl]
                f = 1.0 / (1.0 + jnp.exp(-z))
                out_v[e, sl] = f * g_a[b][e, pl.ds(64 + j * 16, 16)]
            return carry

        lax.fori_loop(0, CHUNK, _edge, 0)

    def fire_gather_b(b, l):
        offset(lin_src[l])
        return [pltpu.async_copy(h2.at[lin_src[l]], g_w[b], sem_g[b])]

    def wait_gather_b(b):
        pltpu.make_async_copy(h2.at[pl.ds(0, CHUNK)], g_w[b],
                              sem_g[b]).wait()

    def compute_b(b, l):
        def _edge(e, carry):
            tv = lin_t[l][e]
            for j in range(4):
                sl = pl.ds(j * 16, 16)
                out_v[e, sl] = g_w[b][e, sl] + tv
            return carry

        lax.fori_loop(0, CHUNK, _edge, 0)

    def run_pass(fire_gather, wait_gather, compute):
        fire_lin(0, 0)
        wait_lin(0)
        fire_gather(0, 0)
        fire_lin(1, 1)
        fire_lin(2, 2)

        def outer(g, carry):
            ci0 = g * 4
            for k in range(4):
                ci = ci0 + k
                b = k % NG
                l = k % NLIN

                @pl.when(ci + 1 < NCHUNKS)
                def _pref():
                    wait_lin((k + 1) % NLIN)
                    fire_gather((k + 1) % NG, (k + 1) % NLIN)

                @pl.when(ci < NCHUNKS)
                def _work():
                    wait_gather(b)
                    compute(b, l)
                    pltpu.sync_copy(out_v, acc.at[lin_dst[l]], add=True)

                @pl.when(ci + 3 < NCHUNKS)
                def _lin():
                    fire_lin(ci + 3, (k + 3) % NLIN)
            return carry

        lax.fori_loop(0, (NCHUNKS + 3) // 4, outer, 0)

    pltpu.sync_copy(svec.at[pl.ds(cid * 64, 64)], s_v)
    zero_stripe()
    plsc.subcore_barrier()
    run_pass(fire_gather_a, wait_gather_a, compute_a)
    plsc.subcore_barrier()
    copy_out(out_c)
    zero_stripe()
    plsc.subcore_barrier()
    run_pass(fire_gather_b, wait_gather_b, compute_b)
    plsc.subcore_barrier()
    copy_out(out_h)


def _make_sc_kernel(interpret=False):
    return pl.kernel(
        _sc_edge_body,
        out_type=[
            jax.ShapeDtypeStruct((2 * N, 64), jnp.float32),
            jax.ShapeDtypeStruct((2 * N, 64), jnp.float32),
        ],
        mesh=_sc_mesh,
        scratch_types=_sc_scratch,
        compiler_params=pltpu.CompilerParams(use_tc_tiling_on_sc=False),
        interpret=interpret,
    )


_sc_edge_kernel = _make_sc_kernel()


def _tc2_body(ht_ref, cred_ref, xwiou_ref, uiout_ref, hnew_ref, cnew_ref):
    iou = (jnp.dot(ht_ref[...], uiout_ref[...],
                   preferred_element_type=jnp.float32)
           + xwiou_ref[...])
    i = jax.nn.sigmoid(iou[:, :H])
    o = jax.nn.sigmoid(iou[:, H:2 * H])
    u = jnp.tanh(iou[:, 2 * H:])
    cn = i * u + cred_ref[...]
    cnew_ref[...] = cn
    hnew_ref[...] = o * jnp.tanh(cn)


def _tc2(h_tild, c_red, xwiou, uiout):
    grid = (N // BN,)
    return pl.pallas_call(
        _tc2_body,
        grid=grid,
        in_specs=[
            pl.BlockSpec((BN, H), lambda i: (i, 0)),
            pl.BlockSpec((BN, H), lambda i: (i, 0)),
            pl.BlockSpec((BN, 3 * H), lambda i: (i, 0)),
            pl.BlockSpec((H, 3 * H), lambda i: (0, 0)),
        ],
        out_specs=[
            pl.BlockSpec((BN, H), lambda i: (i, 0)),
            pl.BlockSpec((BN, H), lambda i: (i, 0)),
        ],
        out_shape=[
            jax.ShapeDtypeStruct((N, H), jnp.float32),
            jax.ShapeDtypeStruct((N, H), jnp.float32),
        ],
    )(h_tild, c_red, xwiou, uiout)


def kernel(x, h, c, edge_index, time, W_iou, U_iou, b_iou, U_f, W_f, b_f):
    x = x.astype(jnp.float32)
    h = h.astype(jnp.float32)
    c = c.astype(jnp.float32)
    src = edge_index[0].astype(jnp.int32)
    dst = edge_index[1].astype(jnp.int32)
    t16 = jnp.broadcast_to(time.astype(jnp.float32), (E, 16))

    wxf, atab, xwiou, s = _tc1(x, h, c, W_f.T, U_f.T, W_iou.T, b_f, b_iou)

    atab2 = atab.reshape(2 * N, 128)
    w2 = wxf.reshape(2 * N, 64)
    h2 = h.reshape(2 * N, 64)
    zeros = jnp.zeros((N, 64), jnp.float32)

    out_c, out_h = _sc_edge_kernel(atab2, w2, h2, src, dst, t16,
                                   s.reshape(H), zeros)
    c_red = jnp.concatenate([out_c[:N], out_c[N:]], axis=1)
    h_tild = jnp.concatenate([out_h[:N], out_h[N:]], axis=1)

    h_new, c_new = _tc2(h_tild, c_red, xwiou, U_iou.T)
    return (h_new, c_new)


# speedup vs baseline: 1.4188x; 1.0117x over previous
import functools

import jax
import jax.numpy as jnp
from jax import lax
from jax.experimental import pallas as pl
from jax.experimental.pallas import tpu as pltpu
from jax.experimental.pallas import tpu_sc as plsc

N = 10000
E = 320000
H = 128
NTILES = 16
CHUNK = 80
EDGES_PER_TILE = E // NTILES
NCHUNKS = EDGES_PER_TILE // CHUNK
STRIPE = 624
TAIL = N - NTILES * STRIPE
TAIL_BASE = NTILES * STRIPE
BN = 1000
NLIN = 4
NG = 2


def _tc1_body(x_ref, h_ref, c_ref, wft_ref, uft_ref, wiout_ref, bf_ref,
              biou_ref, wxf_ref, atab_ref, xwiou_ref, s_ref):
    x = x_ref[...]
    h = h_ref[...]
    cc = c_ref[...]
    wxf_ref[...] = (
        jnp.dot(x, wft_ref[...], preferred_element_type=jnp.float32)
        + bf_ref[...])
    hu = jnp.dot(h, uft_ref[...], preferred_element_type=jnp.float32)
    atab_ref[...] = jnp.concatenate(
        [hu[:, :64], cc[:, :64], hu[:, 64:], cc[:, 64:]], axis=1)
    xwiou_ref[...] = (
        jnp.dot(x, wiout_ref[...], preferred_element_type=jnp.float32)
        + biou_ref[...])
    s_ref[...] = jnp.sum(uft_ref[...], axis=0, keepdims=True)


def _tc1(x, h, c, wft, uft, wiout, bf, biou):
    grid = (N // BN,)
    return pl.pallas_call(
        _tc1_body,
        grid=grid,
        in_specs=[
            pl.BlockSpec((BN, H), lambda i: (i, 0)),
            pl.BlockSpec((BN, H), lambda i: (i, 0)),
            pl.BlockSpec((BN, H), lambda i: (i, 0)),
            pl.BlockSpec((H, H), lambda i: (0, 0)),
            pl.BlockSpec((H, H), lambda i: (0, 0)),
            pl.BlockSpec((H, 3 * H), lambda i: (0, 0)),
            pl.BlockSpec((1, H), lambda i: (0, 0)),
            pl.BlockSpec((1, 3 * H), lambda i: (0, 0)),
        ],
        out_specs=[
            pl.BlockSpec((BN, H), lambda i: (i, 0)),
            pl.BlockSpec((BN, 2 * H), lambda i: (i, 0)),
            pl.BlockSpec((BN, 3 * H), lambda i: (i, 0)),
            pl.BlockSpec((1, H), lambda i: (0, 0)),
        ],
        out_shape=[
            jax.ShapeDtypeStruct((N, H), jnp.float32),
            jax.ShapeDtypeStruct((N, 2 * H), jnp.float32),
            jax.ShapeDtypeStruct((N, 3 * H), jnp.float32),
            jax.ShapeDtypeStruct((1, H), jnp.float32),
        ],
    )(x, h, c, wft, uft, wiout, bf, biou)


_sc_mesh = plsc.VectorSubcoreMesh(core_axis_name="c", subcore_axis_name="s")

_sc_scratch = (
    [pltpu.VMEM((CHUNK,), jnp.int32) for _ in range(NLIN)]
    + [pltpu.VMEM((CHUNK,), jnp.int32) for _ in range(NLIN)]
    + [pltpu.VMEM((CHUNK,), jnp.int32) for _ in range(NG)]
    + [pltpu.VMEM((CHUNK, 16), jnp.float32) for _ in range(NLIN)]
    + [pltpu.VMEM((CHUNK, 2 * 64), jnp.float32) for _ in range(NG)]
    + [pltpu.VMEM((CHUNK, 64), jnp.float32) for _ in range(NG)]
    + [
        pltpu.VMEM((CHUNK, 64), jnp.float32),
        pltpu.VMEM((64,), jnp.float32),
        pltpu.VMEM_SHARED((N, 64), jnp.float32),
        pltpu.VMEM_SHARED((N, 16), jnp.float32),
    ]
    + [pltpu.SemaphoreType.DMA for _ in range(NLIN + NG)]
)


def _sc_edge_body(atab, w2, h2, src, dst, t16, svec, zeros, zeros_t,
                  out_c, out_h, out_t, *scr):
    pos = 0
    lin_src = scr[pos:pos + NLIN]; pos += NLIN
    lin_dst = scr[pos:pos + NLIN]; pos += NLIN
    lin_dsto = scr[pos:pos + NG]; pos += NG
    lin_t = scr[pos:pos + NLIN]; pos += NLIN
    g_a = scr[pos:pos + NG]; pos += NG
    g_w = scr[pos:pos + NG]; pos += NG
    out_v, s_v, acc, acc_t = scr[pos:pos + 4]; pos += 4
    sem_lin = scr[pos:pos + NLIN]; pos += NLIN
    sem_g = scr[pos:pos + NG]; pos += NG

    cid = lax.axis_index("c")
    sid = lax.axis_index("s")
    row0 = sid * STRIPE
    ebase = sid * EDGES_PER_TILE

    def zero_stripe():
        pltpu.sync_copy(zeros.at[pl.ds(row0, STRIPE)],
                        acc.at[pl.ds(row0, STRIPE)])

        @pl.when(sid == NTILES - 1)
        def _zero_tail():
            pltpu.sync_copy(zeros.at[pl.ds(TAIL_BASE, TAIL)],
                            acc.at[pl.ds(TAIL_BASE, TAIL)])

    def copy_out(out):
        pltpu.sync_copy(acc.at[pl.ds(row0, STRIPE)],
                        out.at[pl.ds(cid * N + row0, STRIPE)])

        @pl.when(sid == NTILES - 1)
        def _out_tail():
            pltpu.sync_copy(acc.at[pl.ds(TAIL_BASE, TAIL)],
                            out.at[pl.ds(cid * N + TAIL_BASE, TAIL)])

    def fire_lin(ci, l):
        base = ebase + ci * CHUNK
        pltpu.async_copy(src.at[pl.ds(base, CHUNK)], lin_src[l], sem_lin[l])
        pltpu.async_copy(dst.at[pl.ds(base, CHUNK)], lin_dst[l], sem_lin[l])
        pltpu.async_copy(t16.at[pl.ds(base, CHUNK)], lin_t[l], sem_lin[l])

    def wait_lin(l):
        pltpu.make_async_copy(src.at[pl.ds(0, CHUNK)], lin_src[l],
                              sem_lin[l]).wait()
        pltpu.make_async_copy(dst.at[pl.ds(0, CHUNK)], lin_dst[l],
                              sem_lin[l]).wait()
        pltpu.make_async_copy(t16.at[pl.ds(0, CHUNK)], lin_t[l],
                              sem_lin[l]).wait()

    def offset(ref_v):
        for j in range(CHUNK // 16):
            sl = pl.ds(j * 16, 16)
            ref_v[sl] = ref_v[sl] * 2 + cid

    def fire_gather_a(b, l):
        offset(lin_src[l])
        for j in range(CHUNK // 16):
            sl = pl.ds(j * 16, 16)
            lin_dsto[b][sl] = lin_dst[l][sl] * 2 + cid
        return [pltpu.async_copy(atab.at[lin_src[l]], g_a[b], sem_g[b]),
                pltpu.async_copy(w2.at[lin_dsto[b]], g_w[b], sem_g[b])]

    def wait_gather_a(b):
        pltpu.make_async_copy(atab.at[pl.ds(0, CHUNK)], g_a[b],
                              sem_g[b]).wait()
        pltpu.make_async_copy(w2.at[pl.ds(0, CHUNK)], g_w[b],
                              sem_g[b]).wait()

    def compute_a(b, l):
        def _edge(e, carry):
            tv = lin_t[l][e]
            for j in range(4):
                sl = pl.ds(j * 16, 16)
                z = g_a[b][e, sl] + g_w[b][e, sl] + tv * s_v[sl]
                f = 1.0 / (1.0 + jnp.exp(-z))
                out_v[e, sl] = f * g_a[b][e, pl.ds(64 + j * 16, 16)]
            return carry

        lax.fori_loop(0, CHUNK, _edge, 0)

    def fire_gather_b(b, l):
        offset(lin_src[l])
        return [pltpu.async_copy(h2.at[lin_src[l]], g_w[b], sem_g[b])]

    def wait_gather_b(b):
        pltpu.make_async_copy(h2.at[pl.ds(0, CHUNK)], g_w[b],
                              sem_g[b]).wait()

    def work_a(b, l):
        wait_gather_a(b)
        compute_a(b, l)
        pltpu.sync_copy(out_v, acc.at[lin_dst[l]], add=True)

    def work_b(b, l):
        wait_gather_b(b)
        pltpu.sync_copy(g_w[b], acc.at[lin_dst[l]], add=True)
        pltpu.sync_copy(lin_t[l], acc_t.at[lin_dst[l]], add=True)

    def run_pass(fire_gather, work):
        fire_lin(0, 0)
        wait_lin(0)
        fire_gather(0, 0)
        fire_lin(1, 1)
        fire_lin(2, 2)

        def outer(g, carry):
            ci0 = g * 4
            for k in range(4):
                ci = ci0 + k
                b = k % NG
                l = k % NLIN

                @pl.when(ci + 1 < NCHUNKS)
                def _pref():
                    wait_lin((k + 1) % NLIN)
                    fire_gather((k + 1) % NG, (k + 1) % NLIN)

                @pl.when(ci < NCHUNKS)
                def _work():
                    work(b, l)

                @pl.when(ci + 3 < NCHUNKS)
                def _lin():
                    fire_lin(ci + 3, (k + 3) % NLIN)
            return carry

        lax.fori_loop(0, (NCHUNKS + 3) // 4, outer, 0)

    pltpu.sync_copy(svec.at[pl.ds(cid * 64, 64)], s_v)
    zero_stripe()
    plsc.subcore_barrier()
    run_pass(fire_gather_a, work_a)
    plsc.subcore_barrier()
    copy_out(out_c)
    zero_stripe()
    pltpu.sync_copy(zeros_t.at[pl.ds(row0, STRIPE)],
                    acc_t.at[pl.ds(row0, STRIPE)])

    @pl.when(sid == NTILES - 1)
    def _zero_t_tail():
        pltpu.sync_copy(zeros_t.at[pl.ds(TAIL_BASE, TAIL)],
                        acc_t.at[pl.ds(TAIL_BASE, TAIL)])

    plsc.subcore_barrier()
    run_pass(fire_gather_b, work_b)
    plsc.subcore_barrier()
    copy_out(out_h)

    @pl.when(cid == 0)
    def _copy_t():
        pltpu.sync_copy(acc_t.at[pl.ds(row0, STRIPE)],
                        out_t.at[pl.ds(row0, STRIPE)])

        @pl.when(sid == NTILES - 1)
        def _out_t_tail():
            pltpu.sync_copy(acc_t.at[pl.ds(TAIL_BASE, TAIL)],
                            out_t.at[pl.ds(TAIL_BASE, TAIL)])


def _make_sc_kernel(interpret=False):
    return pl.kernel(
        _sc_edge_body,
        out_type=[
            jax.ShapeDtypeStruct((2 * N, 64), jnp.float32),
            jax.ShapeDtypeStruct((2 * N, 64), jnp.float32),
            jax.ShapeDtypeStruct((N, 16), jnp.float32),
        ],
        mesh=_sc_mesh,
        scratch_types=_sc_scratch,
        compiler_params=pltpu.CompilerParams(use_tc_tiling_on_sc=False),
        interpret=interpret,
    )


_sc_edge_kernel = _make_sc_kernel()


def _tc2_body(ht_ref, tacc_ref, cred_ref, xwiou_ref, uiout_ref,
              hnew_ref, cnew_ref):
    ht = ht_ref[...] + tacc_ref[:, 0:1]
    iou = (jnp.dot(ht, uiout_ref[...],
                   preferred_element_type=jnp.float32)
           + xwiou_ref[...])
    i = jax.nn.sigmoid(iou[:, :H])
    o = jax.nn.sigmoid(iou[:, H:2 * H])
    u = jnp.tanh(iou[:, 2 * H:])
    cn = i * u + cred_ref[...]
    cnew_ref[...] = cn
    hnew_ref[...] = o * jnp.tanh(cn)


def _tc2(h_tild, tacc, c_red, xwiou, uiout):
    grid = (N // BN,)
    return pl.pallas_call(
        _tc2_body,
        grid=grid,
        in_specs=[
            pl.BlockSpec((BN, H), lambda i: (i, 0)),
            pl.BlockSpec((BN, 16), lambda i: (i, 0)),
            pl.BlockSpec((BN, H), lambda i: (i, 0)),
            pl.BlockSpec((BN, 3 * H), lambda i: (i, 0)),
            pl.BlockSpec((H, 3 * H), lambda i: (0, 0)),
        ],
        out_specs=[
            pl.BlockSpec((BN, H), lambda i: (i, 0)),
            pl.BlockSpec((BN, H), lambda i: (i, 0)),
        ],
        out_shape=[
            jax.ShapeDtypeStruct((N, H), jnp.float32),
            jax.ShapeDtypeStruct((N, H), jnp.float32),
        ],
    )(h_tild, tacc, c_red, xwiou, uiout)


def kernel(x, h, c, edge_index, time, W_iou, U_iou, b_iou, U_f, W_f, b_f):
    x = x.astype(jnp.float32)
    h = h.astype(jnp.float32)
    c = c.astype(jnp.float32)
    src = edge_index[0].astype(jnp.int32)
    dst = edge_index[1].astype(jnp.int32)
    t16 = jnp.broadcast_to(time.astype(jnp.float32), (E, 16))

    wxf, atab, xwiou, s = _tc1(x, h, c, W_f.T, U_f.T, W_iou.T, b_f, b_iou)

    atab2 = atab.reshape(2 * N, 128)
    w2 = wxf.reshape(2 * N, 64)
    h2 = h.reshape(2 * N, 64)
    zeros = jnp.zeros((N, 64), jnp.float32)
    zeros_t = jnp.zeros((N, 16), jnp.float32)

    out_c, out_h, out_t = _sc_edge_kernel(atab2, w2, h2, src, dst, t16,
                                          s.reshape(H), zeros, zeros_t)
    c_red = jnp.concatenate([out_c[:N], out_c[N:]], axis=1)
    h_acc = jnp.concatenate([out_h[:N], out_h[N:]], axis=1)

    h_new, c_new = _tc2(h_acc, out_t, c_red, xwiou, U_iou.T)
    return (h_new, c_new)


# speedup vs baseline: 1.4340x; 1.0107x over previous
import functools

import jax
import jax.numpy as jnp
from jax import lax
from jax.experimental import pallas as pl
from jax.experimental.pallas import tpu as pltpu
from jax.experimental.pallas import tpu_sc as plsc

N = 10000
E = 320000
H = 128
NTILES = 16
CHUNK = 80
EDGES_PER_TILE = E // NTILES
NCHUNKS = EDGES_PER_TILE // CHUNK
STRIPE = 624
TAIL = N - NTILES * STRIPE
TAIL_BASE = NTILES * STRIPE
BN = 1000
NLIN = 4
NG = 4


def _tc1_body(x_ref, h_ref, c_ref, wft_ref, uft_ref, wiout_ref, bf_ref,
              biou_ref, wxf_ref, atab_ref, xwiou_ref, s_ref):
    x = x_ref[...]
    h = h_ref[...]
    cc = c_ref[...]
    wxf_ref[...] = (
        jnp.dot(x, wft_ref[...], preferred_element_type=jnp.float32)
        + bf_ref[...])
    hu = jnp.dot(h, uft_ref[...], preferred_element_type=jnp.float32)
    atab_ref[...] = jnp.concatenate(
        [hu[:, :64], cc[:, :64], hu[:, 64:], cc[:, 64:]], axis=1)
    xwiou_ref[...] = (
        jnp.dot(x, wiout_ref[...], preferred_element_type=jnp.float32)
        + biou_ref[...])
    s_ref[...] = jnp.sum(uft_ref[...], axis=0, keepdims=True)


def _tc1(x, h, c, wft, uft, wiout, bf, biou):
    grid = (N // BN,)
    return pl.pallas_call(
        _tc1_body,
        grid=grid,
        in_specs=[
            pl.BlockSpec((BN, H), lambda i: (i, 0)),
            pl.BlockSpec((BN, H), lambda i: (i, 0)),
            pl.BlockSpec((BN, H), lambda i: (i, 0)),
            pl.BlockSpec((H, H), lambda i: (0, 0)),
            pl.BlockSpec((H, H), lambda i: (0, 0)),
            pl.BlockSpec((H, 3 * H), lambda i: (0, 0)),
            pl.BlockSpec((1, H), lambda i: (0, 0)),
            pl.BlockSpec((1, 3 * H), lambda i: (0, 0)),
        ],
        out_specs=[
            pl.BlockSpec((BN, H), lambda i: (i, 0)),
            pl.BlockSpec((BN, 2 * H), lambda i: (i, 0)),
            pl.BlockSpec((BN, 3 * H), lambda i: (i, 0)),
            pl.BlockSpec((1, H), lambda i: (0, 0)),
        ],
        out_shape=[
            jax.ShapeDtypeStruct((N, H), jnp.float32),
            jax.ShapeDtypeStruct((N, 2 * H), jnp.float32),
            jax.ShapeDtypeStruct((N, 3 * H), jnp.float32),
            jax.ShapeDtypeStruct((1, H), jnp.float32),
        ],
    )(x, h, c, wft, uft, wiout, bf, biou)


_sc_mesh = plsc.VectorSubcoreMesh(core_axis_name="c", subcore_axis_name="s")

_sc_scratch = (
    [pltpu.VMEM((CHUNK,), jnp.int32) for _ in range(NLIN)]
    + [pltpu.VMEM((CHUNK,), jnp.int32) for _ in range(NLIN)]
    + [pltpu.VMEM((CHUNK,), jnp.int32) for _ in range(NG)]
    + [pltpu.VMEM((CHUNK, 16), jnp.float32) for _ in range(NLIN)]
    + [pltpu.VMEM((CHUNK, 2 * 64), jnp.float32) for _ in range(NG)]
    + [pltpu.VMEM((CHUNK, 64), jnp.float32) for _ in range(NG)]
    + [
        pltpu.VMEM((CHUNK, 64), jnp.float32),
        pltpu.VMEM((64,), jnp.float32),
        pltpu.VMEM_SHARED((N, 64), jnp.float32),
        pltpu.VMEM_SHARED((N, 16), jnp.float32),
    ]
    + [pltpu.SemaphoreType.DMA for _ in range(NLIN + NG)]
)


def _sc_edge_body(atab, w2, h2, src, dst, t16, svec, zeros, zeros_t,
                  out_c, out_h, out_t, *scr):
    pos = 0
    lin_src = scr[pos:pos + NLIN]; pos += NLIN
    lin_dst = scr[pos:pos + NLIN]; pos += NLIN
    lin_dsto = scr[pos:pos + NG]; pos += NG
    lin_t = scr[pos:pos + NLIN]; pos += NLIN
    g_a = scr[pos:pos + NG]; pos += NG
    g_w = scr[pos:pos + NG]; pos += NG
    out_v, s_v, acc, acc_t = scr[pos:pos + 4]; pos += 4
    sem_lin = scr[pos:pos + NLIN]; pos += NLIN
    sem_g = scr[pos:pos + NG]; pos += NG

    cid = lax.axis_index("c")
    sid = lax.axis_index("s")
    row0 = sid * STRIPE
    ebase = sid * EDGES_PER_TILE

    def zero_stripe():
        pltpu.sync_copy(zeros.at[pl.ds(row0, STRIPE)],
                        acc.at[pl.ds(row0, STRIPE)])

        @pl.when(sid == NTILES - 1)
        def _zero_tail():
            pltpu.sync_copy(zeros.at[pl.ds(TAIL_BASE, TAIL)],
                            acc.at[pl.ds(TAIL_BASE, TAIL)])

    def copy_out(out):
        pltpu.sync_copy(acc.at[pl.ds(row0, STRIPE)],
                        out.at[pl.ds(cid * N + row0, STRIPE)])

        @pl.when(sid == NTILES - 1)
        def _out_tail():
            pltpu.sync_copy(acc.at[pl.ds(TAIL_BASE, TAIL)],
                            out.at[pl.ds(cid * N + TAIL_BASE, TAIL)])

    def fire_lin(ci, l):
        base = ebase + ci * CHUNK
        pltpu.async_copy(src.at[pl.ds(base, CHUNK)], lin_src[l], sem_lin[l])
        pltpu.async_copy(dst.at[pl.ds(base, CHUNK)], lin_dst[l], sem_lin[l])
        pltpu.async_copy(t16.at[pl.ds(base, CHUNK)], lin_t[l], sem_lin[l])

    def wait_lin(l):
        pltpu.make_async_copy(src.at[pl.ds(0, CHUNK)], lin_src[l],
                              sem_lin[l]).wait()
        pltpu.make_async_copy(dst.at[pl.ds(0, CHUNK)], lin_dst[l],
                              sem_lin[l]).wait()
        pltpu.make_async_copy(t16.at[pl.ds(0, CHUNK)], lin_t[l],
                              sem_lin[l]).wait()

    def offset(ref_v):
        for j in range(CHUNK // 16):
            sl = pl.ds(j * 16, 16)
            ref_v[sl] = ref_v[sl] * 2 + cid

    def fire_gather_a(b, l):
        offset(lin_src[l])
        for j in range(CHUNK // 16):
            sl = pl.ds(j * 16, 16)
            lin_dsto[b][sl] = lin_dst[l][sl] * 2 + cid
        return [pltpu.async_copy(atab.at[lin_src[l]], g_a[b], sem_g[b]),
                pltpu.async_copy(w2.at[lin_dsto[b]], g_w[b], sem_g[b])]

    def wait_gather_a(b):
        pltpu.make_async_copy(atab.at[pl.ds(0, CHUNK)], g_a[b],
                              sem_g[b]).wait()
        pltpu.make_async_copy(w2.at[pl.ds(0, CHUNK)], g_w[b],
                              sem_g[b]).wait()

    def compute_a(b, l):
        def _edge(e, carry):
            tv = lin_t[l][e]
            for j in range(4):
                sl = pl.ds(j * 16, 16)
                z = g_a[b][e, sl] + g_w[b][e, sl] + tv * s_v[sl]
                f = 1.0 / (1.0 + jnp.exp(-z))
                out_v[e, sl] = f * g_a[b][e, pl.ds(64 + j * 16, 16)]
            return carry

        lax.fori_loop(0, CHUNK, _edge, 0)

    def fire_gather_b(b, l):
        offset(lin_src[l])
        return [pltpu.async_copy(h2.at[lin_src[l]], g_w[b], sem_g[b])]

    def wait_gather_b(b):
        pltpu.make_async_copy(h2.at[pl.ds(0, CHUNK)], g_w[b],
                              sem_g[b]).wait()

    def work_a(b, l):
        wait_gather_a(b)
        compute_a(b, l)
        pltpu.sync_copy(out_v, acc.at[lin_dst[l]], add=True)

    def work_b(b, l):
        wait_gather_b(b)
        pltpu.sync_copy(g_w[b], acc.at[lin_dst[l]], add=True)
        pltpu.sync_copy(lin_t[l], acc_t.at[lin_dst[l]], add=True)

    def run_pass(fire_gather, work):
        fire_lin(0, 0)
        fire_lin(1, 1)
        wait_lin(0)
        fire_gather(0, 0)
        fire_lin(2, 2)
        wait_lin(1)
        fire_gather(1, 1)
        fire_lin(3, 3)

        def outer(g, carry):
            ci0 = g * 4
            for k in range(4):
                ci = ci0 + k

                @pl.when(ci + 2 < NCHUNKS)
                def _pref():
                    wait_lin((k + 2) % NLIN)
                    fire_gather((k + 2) % NG, (k + 2) % NLIN)

                @pl.when(ci < NCHUNKS)
                def _work():
                    work(k, k)

                @pl.when(ci + 4 < NCHUNKS)
                def _lin():
                    fire_lin(ci + 4, k)
            return carry

        lax.fori_loop(0, (NCHUNKS + 3) // 4, outer, 0)

    pltpu.sync_copy(svec.at[pl.ds(cid * 64, 64)], s_v)
    zero_stripe()
    plsc.subcore_barrier()
    run_pass(fire_gather_a, work_a)
    plsc.subcore_barrier()
    copy_out(out_c)
    zero_stripe()
    pltpu.sync_copy(zeros_t.at[pl.ds(row0, STRIPE)],
                    acc_t.at[pl.ds(row0, STRIPE)])

    @pl.when(sid == NTILES - 1)
    def _zero_t_tail():
        pltpu.sync_copy(zeros_t.at[pl.ds(TAIL_BASE, TAIL)],
                        acc_t.at[pl.ds(TAIL_BASE, TAIL)])

    plsc.subcore_barrier()
    run_pass(fire_gather_b, work_b)
    plsc.subcore_barrier()
    copy_out(out_h)

    @pl.when(cid == 0)
    def _copy_t():
        pltpu.sync_copy(acc_t.at[pl.ds(row0, STRIPE)],
                        out_t.at[pl.ds(row0, STRIPE)])

        @pl.when(sid == NTILES - 1)
        def _out_t_tail():
            pltpu.sync_copy(acc_t.at[pl.ds(TAIL_BASE, TAIL)],
                            out_t.at[pl.ds(TAIL_BASE, TAIL)])


def _make_sc_kernel(interpret=False):
    return pl.kernel(
        _sc_edge_body,
        out_type=[
            jax.ShapeDtypeStruct((2 * N, 64), jnp.float32),
            jax.ShapeDtypeStruct((2 * N, 64), jnp.float32),
            jax.ShapeDtypeStruct((N, 16), jnp.float32),
        ],
        mesh=_sc_mesh,
        scratch_types=_sc_scratch,
        compiler_params=pltpu.CompilerParams(use_tc_tiling_on_sc=False),
        interpret=interpret,
    )


_sc_edge_kernel = _make_sc_kernel()


def _tc2_body(ht_ref, tacc_ref, cred_ref, xwiou_ref, uiout_ref,
              hnew_ref, cnew_ref):
    ht = ht_ref[...] + tacc_ref[:, 0:1]
    iou = (jnp.dot(ht, uiout_ref[...],
                   preferred_element_type=jnp.float32)
           + xwiou_ref[...])
    i = jax.nn.sigmoid(iou[:, :H])
    o = jax.nn.sigmoid(iou[:, H:2 * H])
    u = jnp.tanh(iou[:, 2 * H:])
    cn = i * u + cred_ref[...]
    cnew_ref[...] = cn
    hnew_ref[...] = o * jnp.tanh(cn)


def _tc2(h_tild, tacc, c_red, xwiou, uiout):
    grid = (N // BN,)
    return pl.pallas_call(
        _tc2_body,
        grid=grid,
        in_specs=[
            pl.BlockSpec((BN, H), lambda i: (i, 0)),
            pl.BlockSpec((BN, 16), lambda i: (i, 0)),
            pl.BlockSpec((BN, H), lambda i: (i, 0)),
            pl.BlockSpec((BN, 3 * H), lambda i: (i, 0)),
            pl.BlockSpec((H, 3 * H), lambda i: (0, 0)),
        ],
        out_specs=[
            pl.BlockSpec((BN, H), lambda i: (i, 0)),
            pl.BlockSpec((BN, H), lambda i: (i, 0)),
        ],
        out_shape=[
            jax.ShapeDtypeStruct((N, H), jnp.float32),
            jax.ShapeDtypeStruct((N, H), jnp.float32),
        ],
    )(h_tild, tacc, c_red, xwiou, uiout)


def kernel(x, h, c, edge_index, time, W_iou, U_iou, b_iou, U_f, W_f, b_f):
    x = x.astype(jnp.float32)
    h = h.astype(jnp.float32)
    c = c.astype(jnp.float32)
    src = edge_index[0].astype(jnp.int32)
    dst = edge_index[1].astype(jnp.int32)
    t16 = jnp.broadcast_to(time.astype(jnp.float32), (E, 16))

    wxf, atab, xwiou, s = _tc1(x, h, c, W_f.T, U_f.T, W_iou.T, b_f, b_iou)

    atab2 = atab.reshape(2 * N, 128)
    w2 = wxf.reshape(2 * N, 64)
    h2 = h.reshape(2 * N, 64)
    zeros = jnp.zeros((N, 64), jnp.float32)
    zeros_t = jnp.zeros((N, 16), jnp.float32)

    out_c, out_h, out_t = _sc_edge_kernel(atab2, w2, h2, src, dst, t16,
                                          s.reshape(H), zeros, zeros_t)
    c_red = jnp.concatenate([out_c[:N], out_c[N:]], axis=1)
    h_acc = jnp.concatenate([out_h[:N], out_h[N:]], axis=1)

    h_new, c_new = _tc2(h_acc, out_t, c_red, xwiou, U_iou.T)
    return (h_new, c_new)
